# baseline (speedup 1.0000x reference)
import functools

import jax
import jax.numpy as jnp
from jax import lax
from jax.experimental import pallas as pl
from jax.experimental.pallas import tpu as pltpu
from jax.experimental.pallas import tpu_sc as plsc

B = 16384
D = 32
NC = 2
NS = 16
NW = NC * NS
BPW = B // NW
CHUNK = 128
NCHUNK = BPW // CHUNK


def _mf_body(users_hbm, items_hbm, ut_hbm, it_hbm, out_hbm,
             uidx_v, iidx_v, urows_v, irows_v, stage_v, out_v, sem_u, sem_i):
    wid = lax.axis_index("s") * NC + lax.axis_index("c")
    base = wid * BPW

    pltpu.sync_copy(users_hbm.at[wid], uidx_v)
    pltpu.sync_copy(items_hbm.at[wid], iidx_v)

    for j in range(NCHUNK):
        pltpu.async_copy(ut_hbm.at[uidx_v.at[j]],
                         urows_v.at[pl.ds(j * CHUNK, CHUNK)], sem_u)
        pltpu.async_copy(it_hbm.at[iidx_v.at[j]],
                         irows_v.at[pl.ds(j * CHUNK, CHUNK)], sem_i)
    for j in range(NCHUNK):
        pltpu.make_async_copy(ut_hbm.at[uidx_v.at[j]],
                              urows_v.at[pl.ds(j * CHUNK, CHUNK)], sem_u).wait()
        pltpu.make_async_copy(it_hbm.at[iidx_v.at[j]],
                              irows_v.at[pl.ds(j * CHUNK, CHUNK)], sem_i).wait()

    def row(b, carry):
        u0 = urows_v[b, pl.ds(0, 16)]
        u1 = urows_v[b, pl.ds(16, 16)]
        i0 = irows_v[b, pl.ds(0, 16)]
        i1 = irows_v[b, pl.ds(16, 16)]
        stage_v[pl.ds(b * 16, 16)] = u0 * i0 + u1 * i1
        return carry

    lax.fori_loop(0, BPW, row, 0, unroll=8)

    lanes = lax.iota(jnp.int32, 16)

    def group(k, carry):
        base = k * 256 + lanes * 16
        acc = plsc.load_gather(stage_v, [base])
        for l in range(1, 16):
            acc = acc + plsc.load_gather(stage_v, [base + l])
        x = jnp.minimum(jnp.maximum(acc, -10.0), 10.0)
        out_v[pl.ds(k * 16, 16)] = 1.0 / (1.0 + jnp.exp(-x))
        return carry

    lax.fori_loop(0, BPW // 16, group, 0, unroll=2)

    pltpu.sync_copy(out_v, out_hbm.at[pl.ds(base, BPW)])


@jax.jit
def _mf_forward(users_r, items_r, user_table, item_table):
    mesh = plsc.VectorSubcoreMesh(core_axis_name="c", subcore_axis_name="s")
    return pl.kernel(
        _mf_body,
        mesh=mesh,
        out_type=jax.ShapeDtypeStruct((B,), jnp.float32),
        compiler_params=pltpu.CompilerParams(
            needs_layout_passes=False, use_tc_tiling_on_sc=False),
        scratch_types=[
            pltpu.VMEM((NCHUNK, CHUNK), jnp.int32),
            pltpu.VMEM((NCHUNK, CHUNK), jnp.int32),
            pltpu.VMEM((BPW, D), jnp.float32),
            pltpu.VMEM((BPW, D), jnp.float32),
            pltpu.VMEM((BPW * 16,), jnp.float32),
            pltpu.VMEM((BPW,), jnp.float32),
            pltpu.SemaphoreType.DMA,
            pltpu.SemaphoreType.DMA,
        ],
    )(users_r, items_r, user_table, item_table)


def kernel(users, items, user_table, item_table):
    users_r = jnp.reshape(users.astype(jnp.int32), (NW, NCHUNK, CHUNK))
    items_r = jnp.reshape(items.astype(jnp.int32), (NW, NCHUNK, CHUNK))
    return _mf_forward(users_r, items_r, user_table, item_table)
